# Initial kernel scaffold; baseline (speedup 1.0000x reference)
import jax
import jax.numpy as jnp
from jax.experimental import pallas as pl


def kernel(x, ei, W_e1, b_e1, W_e2, b_e2, W_a1, b_a1, W_a2, b_a2, W_s, b_s):
    raise NotImplementedError("write your pallas kernel here")


# speedup vs baseline: 12.2899x; 12.2899x over previous
import functools

import jax
import jax.numpy as jnp
from jax import lax
from jax.experimental import pallas as pl
from jax.experimental.pallas import tpu as pltpu
from jax.experimental.pallas import tpu_sc as plsc

N = 10000
E = 160000
NC, NS = 2, 16
CH = 128
EPC = E // NC
CPC = EPC // CH
ZCH = 80
NZ = N // ZCH
BN = 1000


@functools.lru_cache(maxsize=None)
def _make_spmm(D):
    mesh = plsc.VectorSubcoreMesh(core_axis_name="c", subcore_axis_name="s")

    @functools.partial(
        pl.kernel,
        out_type=jax.ShapeDtypeStruct((NC, N, D), jnp.float32),
        mesh=mesh,
        compiler_params=pltpu.CompilerParams(use_tc_tiling_on_sc=False),
        scratch_types=[
            pltpu.VMEM((CH,), jnp.int32),
            pltpu.VMEM((CH,), jnp.int32),
            pltpu.VMEM((CH, D), jnp.float32),
            pltpu.VMEM((ZCH, D), jnp.float32),
            pltpu.VMEM_SHARED((N, D), jnp.float32),
            pltpu.SemaphoreType.DMA,
        ],
    )
    def spmm(h_hbm, row_hbm, col_hbm, out_hbm, ridx, cidx, rows, zbuf, acc, sem):
        c = lax.axis_index("c")
        s = lax.axis_index("s")

        zv = jnp.zeros((16,), jnp.float32)

        def zrow(i, carry):
            for d in range(D // 16):
                zbuf[i, pl.ds(d * 16, 16)] = zv
            return carry

        lax.fori_loop(0, ZCH, zrow, 0)

        def zacc(j, carry):
            k = s + j * NS

            @pl.when(k < NZ)
            def _():
                pltpu.sync_copy(zbuf, acc.at[pl.ds(k * ZCH, ZCH)])

            return carry

        lax.fori_loop(0, (NZ + NS - 1) // NS, zacc, 0)
        plsc.subcore_barrier()

        def edge(j, carry):
            k = s + j * NS

            @pl.when(k < CPC)
            def _():
                off = c * EPC + k * CH
                pltpu.sync_copy(row_hbm.at[pl.ds(off, CH)], ridx)
                pltpu.sync_copy(col_hbm.at[pl.ds(off, CH)], cidx)
                pltpu.async_copy(h_hbm.at[ridx], rows, sem).wait()
                pltpu.sync_copy(rows, acc.at[cidx], add=True)

            return carry

        lax.fori_loop(0, (CPC + NS - 1) // NS, edge, 0)
        plsc.subcore_barrier()

        def wout(j, carry):
            k = s + j * NS

            @pl.when(k < NZ)
            def _():
                pltpu.sync_copy(acc.at[pl.ds(k * ZCH, ZCH)],
                                out_hbm.at[c, pl.ds(k * ZCH, ZCH)])

            return carry

        lax.fori_loop(0, (NZ + NS - 1) // NS, wout, 0)

    return spmm


def _row_spec(d):
    return pl.BlockSpec((BN, d), lambda i: (i, 0))


def _acc_spec(d):
    return pl.BlockSpec((NC, BN, d), lambda i: (0, i, 0))


def _full_spec(r, c):
    return pl.BlockSpec((r, c), lambda i: (0, 0))


def _stage_a(dacc, x, w):
    def body(dacc_ref, x_ref, w_ref, dinv_ref, p_ref):
        deg = dacc_ref[0, :, 0:1] + dacc_ref[1, :, 0:1] + 1.0
        dinv = lax.rsqrt(jnp.maximum(deg, 1e-12))
        dinv_ref[...] = dinv
        p_ref[...] = dinv * jnp.dot(x_ref[...], w_ref[...],
                                    preferred_element_type=jnp.float32)

    return pl.pallas_call(
        body,
        grid=(N // BN,),
        in_specs=[_acc_spec(16), _row_spec(256), _full_spec(256, 128)],
        out_specs=[_row_spec(1), _row_spec(128)],
        out_shape=[jax.ShapeDtypeStruct((N, 1), jnp.float32),
                   jax.ShapeDtypeStruct((N, 128), jnp.float32)],
    )(dacc, x, w)


def _stage_b(acc, p, dinv, b, w):
    def body(acc_ref, p_ref, dinv_ref, b_ref, w_ref, o_ref):
        dinv = dinv_ref[...]
        h = jnp.maximum(dinv * (acc_ref[0] + acc_ref[1] + p_ref[...])
                        + b_ref[...], 0.0)
        o_ref[...] = dinv * jnp.dot(h, w_ref[...],
                                    preferred_element_type=jnp.float32)

    return pl.pallas_call(
        body,
        grid=(N // BN,),
        in_specs=[_acc_spec(128), _row_spec(128), _row_spec(1),
                  _full_spec(1, 128), _full_spec(128, 64)],
        out_specs=_row_spec(64),
        out_shape=jax.ShapeDtypeStruct((N, 64), jnp.float32),
    )(acc, p, dinv, b, w)


def _stage_c(acc, p, dinv, b):
    def body(acc_ref, p_ref, dinv_ref, b_ref, o_ref):
        dinv = dinv_ref[...]
        z = jnp.maximum(dinv * (acc_ref[0] + acc_ref[1] + p_ref[...])
                        + b_ref[...], 0.0)
        o_ref[...] = dinv * z

    return pl.pallas_call(
        body,
        grid=(N // BN,),
        in_specs=[_acc_spec(64), _row_spec(64), _row_spec(1),
                  _full_spec(1, 64)],
        out_specs=_row_spec(64),
        out_shape=jax.ShapeDtypeStruct((N, 64), jnp.float32),
    )(acc, p, dinv, b)


def _stage_d(acc, p, dinv, w_a1, b_a1, w_s, b_s):
    def body(acc_ref, p_ref, dinv_ref, wa_ref, ba_ref, ws_ref, bs_ref,
             struct_ref, p4_ref):
        dinv = dinv_ref[...]
        t = dinv * (acc_ref[0] + acc_ref[1] + p_ref[...])
        struct_ref[...] = jnp.maximum(
            jnp.dot(t, ws_ref[...], preferred_element_type=jnp.float32)
            + bs_ref[...], 0.0)
        a = jnp.maximum(
            jnp.dot(t, wa_ref[...], preferred_element_type=jnp.float32)
            + ba_ref[...], 0.0)
        p4_ref[...] = dinv * a

    return pl.pallas_call(
        body,
        grid=(N // BN,),
        in_specs=[_acc_spec(64), _row_spec(64), _row_spec(1),
                  _full_spec(64, 128), _full_spec(1, 128),
                  _full_spec(64, 64), _full_spec(1, 64)],
        out_specs=[_row_spec(64), _row_spec(128)],
        out_shape=[jax.ShapeDtypeStruct((N, 64), jnp.float32),
                   jax.ShapeDtypeStruct((N, 128), jnp.float32)],
    )(acc, p, dinv, w_a1, b_a1, w_s, b_s)


def _stage_e(acc, p, dinv, w, b):
    def body(acc_ref, p_ref, dinv_ref, w_ref, b_ref, o_ref):
        t = dinv_ref[...] * (acc_ref[0] + acc_ref[1] + p_ref[...])
        o_ref[...] = (jnp.dot(t, w_ref[...],
                              preferred_element_type=jnp.float32)
                      + b_ref[...])

    return pl.pallas_call(
        body,
        grid=(N // BN,),
        in_specs=[_acc_spec(128), _row_spec(128), _row_spec(1),
                  _full_spec(128, 256), _full_spec(1, 256)],
        out_specs=_row_spec(256),
        out_shape=jax.ShapeDtypeStruct((N, 256), jnp.float32),
    )(acc, p, dinv, w, b)


def kernel(x, ei, W_e1, b_e1, W_e2, b_e2, W_a1, b_a1, W_a2, b_a2, W_s, b_s):
    row = ei[0].astype(jnp.int32)
    col = ei[1].astype(jnp.int32)

    dacc = _make_spmm(16)(jnp.ones((N, 16), jnp.float32), row, col)

    dinv, p1 = _stage_a(dacc, x, W_e1)
    acc1 = _make_spmm(128)(p1, row, col)
    p2 = _stage_b(acc1, p1, dinv, b_e1.reshape(1, -1), W_e2)
    acc2 = _make_spmm(64)(p2, row, col)
    pz = _stage_c(acc2, p2, dinv, b_e2.reshape(1, -1))
    acc3 = _make_spmm(64)(pz, row, col)
    struct, p4 = _stage_d(acc3, pz, dinv, W_a1, b_a1.reshape(1, -1),
                          W_s, b_s.reshape(1, -1))
    acc4 = _make_spmm(128)(p4, row, col)
    attr = _stage_e(acc4, p4, dinv, W_a2, b_a2.reshape(1, -1))
    return (attr, struct)


# speedup vs baseline: 25.1119x; 2.0433x over previous
import functools

import jax
import jax.numpy as jnp
from jax import lax
from jax.experimental import pallas as pl
from jax.experimental.pallas import tpu as pltpu
from jax.experimental.pallas import tpu_sc as plsc

N = 10000
E = 160000
NC, NS = 2, 16
CH = 128
EPC = E // NC
CPC = EPC // CH
BN = 1000


MAXJ = CPC // NS + 1
NB = 2
NPS = N // NS
ZCH = 25
NZPS = NPS // ZCH


def _span(s):
    lo = (s * CPC) // NS
    hi = ((s + 1) * CPC) // NS
    base = jnp.minimum(lo, CPC - MAXJ)
    return lo - base, hi - base, base


@functools.lru_cache(maxsize=None)
def _make_spmm(D):
    mesh = plsc.VectorSubcoreMesh(core_axis_name="c", subcore_axis_name="s")

    @functools.partial(
        pl.kernel,
        out_type=jax.ShapeDtypeStruct((NC, N, D), jnp.float32),
        mesh=mesh,
        compiler_params=pltpu.CompilerParams(use_tc_tiling_on_sc=False),
        scratch_types=[
            pltpu.VMEM((MAXJ, CH), jnp.int32),
            pltpu.VMEM((MAXJ, CH), jnp.int32),
            [pltpu.VMEM((CH, D), jnp.float32) for _ in range(NB)],
            pltpu.VMEM((ZCH, D), jnp.float32),
            pltpu.VMEM_SHARED((N, D), jnp.float32),
            [pltpu.SemaphoreType.DMA for _ in range(NB)],
            pltpu.SemaphoreType.DMA,
        ],
    )
    def spmm(h_hbm, row_hbm, col_hbm, out_hbm, ridx, cidx, rows, zbuf, acc,
             semg, semi):
        c = lax.axis_index("c")
        s = lax.axis_index("s")
        jlo, jhi, base = _span(s)

        pltpu.async_copy(row_hbm.at[pl.ds(c * CPC + base, MAXJ)], ridx, semi)
        pltpu.sync_copy(col_hbm.at[pl.ds(c * CPC + base, MAXJ)], cidx)
        pltpu.make_async_copy(row_hbm.at[pl.ds(0, MAXJ)], ridx, semi).wait()

        for b in range(NB):
            m = jlo + b

            @pl.when(m < jhi)
            def _(m=m, b=b):
                pltpu.async_copy(h_hbm.at[ridx.at[m]], rows[b], semg[b])

        zv = jnp.zeros((16,), jnp.float32)

        def zrow(i, carry):
            for d in range(D // 16):
                zbuf[i, pl.ds(d * 16, 16)] = zv
            return carry

        lax.fori_loop(0, ZCH, zrow, 0)
        for i in range(NZPS):
            pltpu.sync_copy(zbuf, acc.at[pl.ds(s * NPS + i * ZCH, ZCH)])
        plsc.subcore_barrier()

        def group(g, carry):
            for b in range(NB):
                m = jlo + g * NB + b

                @pl.when(m < jhi)
                def _(m=m, b=b):
                    pltpu.make_async_copy(h_hbm.at[ridx.at[m]], rows[b],
                                          semg[b]).wait()
                    pltpu.sync_copy(rows[b], acc.at[cidx.at[m]], add=True)
                    nxt = m + NB

                    @pl.when(nxt < jhi)
                    def _():
                        pltpu.async_copy(h_hbm.at[ridx.at[nxt]], rows[b],
                                         semg[b])

            return carry

        lax.fori_loop(0, (MAXJ + NB - 1) // NB, group, 0)
        plsc.subcore_barrier()

        pltpu.sync_copy(acc.at[pl.ds(s * NPS, NPS)],
                        out_hbm.at[c, pl.ds(s * NPS, NPS)])

    return spmm


@functools.lru_cache(maxsize=None)
def _make_deg():
    D = 16
    mesh = plsc.VectorSubcoreMesh(core_axis_name="c", subcore_axis_name="s")

    @functools.partial(
        pl.kernel,
        out_type=jax.ShapeDtypeStruct((NC, N, D), jnp.float32),
        mesh=mesh,
        compiler_params=pltpu.CompilerParams(use_tc_tiling_on_sc=False),
        scratch_types=[
            pltpu.VMEM((MAXJ, CH), jnp.int32),
            pltpu.VMEM((CH, D), jnp.float32),
            pltpu.VMEM((ZCH, D), jnp.float32),
            pltpu.VMEM_SHARED((N, D), jnp.float32),
            pltpu.SemaphoreType.DMA,
        ],
    )
    def deg(col_hbm, out_hbm, cidx, ones, zbuf, acc, sems):
        c = lax.axis_index("c")
        s = lax.axis_index("s")
        jlo, jhi, base = _span(s)

        pltpu.sync_copy(col_hbm.at[pl.ds(c * CPC + base, MAXJ)], cidx)

        ov = jnp.ones((16,), jnp.float32)
        zv = jnp.zeros((16,), jnp.float32)

        def orow(i, carry):
            ones[i, pl.ds(0, 16)] = ov
            return carry

        lax.fori_loop(0, CH, orow, 0)

        def zrow(i, carry):
            zbuf[i, pl.ds(0, 16)] = zv
            return carry

        lax.fori_loop(0, ZCH, zrow, 0)
        for i in range(NZPS):
            pltpu.sync_copy(zbuf, acc.at[pl.ds(s * NPS + i * ZCH, ZCH)])
        plsc.subcore_barrier()

        def fire(m, carry):
            @pl.when(m < jhi)
            def _():
                pltpu.async_copy(ones, acc.at[cidx.at[m]], sems, add=True)

            return carry

        lax.fori_loop(jlo, jlo + MAXJ, fire, 0)

        def drain(m, carry):
            @pl.when(m < jhi)
            def _():
                pltpu.make_async_copy(ones, acc.at[cidx.at[m]], sems).wait()

            return carry

        lax.fori_loop(jlo, jlo + MAXJ, drain, 0)
        plsc.subcore_barrier()

        pltpu.sync_copy(acc.at[pl.ds(s * NPS, NPS)],
                        out_hbm.at[c, pl.ds(s * NPS, NPS)])

    return deg


def _row_spec(d):
    return pl.BlockSpec((BN, d), lambda i: (i, 0))


def _acc_spec(d):
    return pl.BlockSpec((NC, BN, d), lambda i: (0, i, 0))


def _full_spec(r, c):
    return pl.BlockSpec((r, c), lambda i: (0, 0))


def _stage_a(dacc, x, w):
    def body(dacc_ref, x_ref, w_ref, dinv_ref, p_ref):
        deg = dacc_ref[0, :, 0:1] + dacc_ref[1, :, 0:1] + 1.0
        dinv = lax.rsqrt(jnp.maximum(deg, 1e-12))
        dinv_ref[...] = dinv
        p_ref[...] = dinv * jnp.dot(x_ref[...], w_ref[...],
                                    preferred_element_type=jnp.float32)

    return pl.pallas_call(
        body,
        grid=(N // BN,),
        in_specs=[_acc_spec(16), _row_spec(256), _full_spec(256, 128)],
        out_specs=[_row_spec(1), _row_spec(128)],
        out_shape=[jax.ShapeDtypeStruct((N, 1), jnp.float32),
                   jax.ShapeDtypeStruct((N, 128), jnp.float32)],
    )(dacc, x, w)


def _stage_b(acc, p, dinv, b, w):
    def body(acc_ref, p_ref, dinv_ref, b_ref, w_ref, o_ref):
        dinv = dinv_ref[...]
        h = jnp.maximum(dinv * (acc_ref[0] + acc_ref[1] + p_ref[...])
                        + b_ref[...], 0.0)
        o_ref[...] = dinv * jnp.dot(h, w_ref[...],
                                    preferred_element_type=jnp.float32)

    return pl.pallas_call(
        body,
        grid=(N // BN,),
        in_specs=[_acc_spec(128), _row_spec(128), _row_spec(1),
                  _full_spec(1, 128), _full_spec(128, 64)],
        out_specs=_row_spec(64),
        out_shape=jax.ShapeDtypeStruct((N, 64), jnp.float32),
    )(acc, p, dinv, b, w)


def _stage_c(acc, p, dinv, b):
    def body(acc_ref, p_ref, dinv_ref, b_ref, o_ref):
        dinv = dinv_ref[...]
        z = jnp.maximum(dinv * (acc_ref[0] + acc_ref[1] + p_ref[...])
                        + b_ref[...], 0.0)
        o_ref[...] = dinv * z

    return pl.pallas_call(
        body,
        grid=(N // BN,),
        in_specs=[_acc_spec(64), _row_spec(64), _row_spec(1),
                  _full_spec(1, 64)],
        out_specs=_row_spec(64),
        out_shape=jax.ShapeDtypeStruct((N, 64), jnp.float32),
    )(acc, p, dinv, b)


def _stage_d(acc, p, dinv, w_a1, b_a1, w_s, b_s):
    def body(acc_ref, p_ref, dinv_ref, wa_ref, ba_ref, ws_ref, bs_ref,
             struct_ref, p4_ref):
        dinv = dinv_ref[...]
        t = dinv * (acc_ref[0] + acc_ref[1] + p_ref[...])
        struct_ref[...] = jnp.maximum(
            jnp.dot(t, ws_ref[...], preferred_element_type=jnp.float32)
            + bs_ref[...], 0.0)
        a = jnp.maximum(
            jnp.dot(t, wa_ref[...], preferred_element_type=jnp.float32)
            + ba_ref[...], 0.0)
        p4_ref[...] = dinv * a

    return pl.pallas_call(
        body,
        grid=(N // BN,),
        in_specs=[_acc_spec(64), _row_spec(64), _row_spec(1),
                  _full_spec(64, 128), _full_spec(1, 128),
                  _full_spec(64, 64), _full_spec(1, 64)],
        out_specs=[_row_spec(64), _row_spec(128)],
        out_shape=[jax.ShapeDtypeStruct((N, 64), jnp.float32),
                   jax.ShapeDtypeStruct((N, 128), jnp.float32)],
    )(acc, p, dinv, w_a1, b_a1, w_s, b_s)


def _stage_e(acc, p, dinv, w, b):
    def body(acc_ref, p_ref, dinv_ref, w_ref, b_ref, o_ref):
        t = dinv_ref[...] * (acc_ref[0] + acc_ref[1] + p_ref[...])
        o_ref[...] = (jnp.dot(t, w_ref[...],
                              preferred_element_type=jnp.float32)
                      + b_ref[...])

    return pl.pallas_call(
        body,
        grid=(N // BN,),
        in_specs=[_acc_spec(128), _row_spec(128), _row_spec(1),
                  _full_spec(128, 256), _full_spec(1, 256)],
        out_specs=_row_spec(256),
        out_shape=jax.ShapeDtypeStruct((N, 256), jnp.float32),
    )(acc, p, dinv, w, b)


def kernel(x, ei, W_e1, b_e1, W_e2, b_e2, W_a1, b_a1, W_a2, b_a2, W_s, b_s):
    row = ei[0].astype(jnp.int32).reshape(NC * CPC, CH)
    col = ei[1].astype(jnp.int32).reshape(NC * CPC, CH)

    dacc = _make_deg()(col)

    dinv, p1 = _stage_a(dacc, x, W_e1)
    acc1 = _make_spmm(128)(p1, row, col)
    p2 = _stage_b(acc1, p1, dinv, b_e1.reshape(1, -1), W_e2)
    acc2 = _make_spmm(64)(p2, row, col)
    pz = _stage_c(acc2, p2, dinv, b_e2.reshape(1, -1))
    acc3 = _make_spmm(64)(pz, row, col)
    struct, p4 = _stage_d(acc3, pz, dinv, W_a1, b_a1.reshape(1, -1),
                          W_s, b_s.reshape(1, -1))
    acc4 = _make_spmm(128)(p4, row, col)
    attr = _stage_e(acc4, p4, dinv, W_a2, b_a2.reshape(1, -1))
    return (attr, struct)


# speedup vs baseline: 25.8923x; 1.0311x over previous
import functools

import jax
import jax.numpy as jnp
from jax import lax
from jax.experimental import pallas as pl
from jax.experimental.pallas import tpu as pltpu
from jax.experimental.pallas import tpu_sc as plsc

N = 10000
E = 160000
NC, NS = 2, 16
CH = 128
EPC = E // NC
CPC = EPC // CH
BN = 2000


MAXJ = CPC // NS + 1
NPS = N // NS
ZCH = 25
NZPS = NPS // ZCH


def _span(s):
    lo = (s * CPC) // NS
    hi = ((s + 1) * CPC) // NS
    base = jnp.minimum(lo, CPC - MAXJ)
    return lo - base, hi - base, base


@functools.lru_cache(maxsize=None)
def _make_spmm(D):
    NB = 2 if D > 64 else 3
    mesh = plsc.VectorSubcoreMesh(core_axis_name="c", subcore_axis_name="s")

    @functools.partial(
        pl.kernel,
        out_type=jax.ShapeDtypeStruct((NC, N, D), jnp.float32),
        mesh=mesh,
        compiler_params=pltpu.CompilerParams(use_tc_tiling_on_sc=False),
        scratch_types=[
            pltpu.VMEM((MAXJ, CH), jnp.int32),
            pltpu.VMEM((MAXJ, CH), jnp.int32),
            [pltpu.VMEM((CH, D), jnp.float32) for _ in range(NB)],
            pltpu.VMEM((ZCH, D), jnp.float32),
            pltpu.VMEM_SHARED((N, D), jnp.float32),
            [pltpu.SemaphoreType.DMA for _ in range(NB)],
            [pltpu.SemaphoreType.DMA for _ in range(NB)],
            pltpu.SemaphoreType.DMA,
        ],
    )
    def spmm(h_hbm, eir_hbm, out_hbm, ridx, cidx, rows, zbuf, acc,
             semg, sems, semi):
        c = lax.axis_index("c")
        s = lax.axis_index("s")
        jlo, jhi, base = _span(s)

        pltpu.async_copy(eir_hbm.at[0, pl.ds(c * CPC + base, MAXJ)], ridx, semi)
        pltpu.sync_copy(eir_hbm.at[1, pl.ds(c * CPC + base, MAXJ)], cidx)
        pltpu.make_async_copy(eir_hbm.at[0, pl.ds(0, MAXJ)], ridx, semi).wait()

        for b in range(NB - 1):
            m = jlo + b

            @pl.when(m < jhi)
            def _(m=m, b=b):
                pltpu.async_copy(h_hbm.at[ridx.at[m]], rows[b], semg[b])

        zv = jnp.zeros((16,), jnp.float32)

        def zrow(i, carry):
            for d in range(D // 16):
                zbuf[i, pl.ds(d * 16, 16)] = zv
            return carry

        lax.fori_loop(0, ZCH, zrow, 0)
        for i in range(NZPS):
            pltpu.async_copy(zbuf, acc.at[pl.ds(s * NPS + i * ZCH, ZCH)], semi)
        for i in range(NZPS):
            pltpu.make_async_copy(zbuf, acc.at[pl.ds(s * NPS + i * ZCH, ZCH)],
                                  semi).wait()
        plsc.subcore_barrier()

        def group(g, carry):
            for b in range(NB):
                m = jlo + g * NB + b
                bp = (b - 1) % NB

                @pl.when(m < jhi)
                def _(m=m, b=b, bp=bp):
                    pltpu.make_async_copy(h_hbm.at[ridx.at[m]], rows[b],
                                          semg[b]).wait()
                    pltpu.async_copy(rows[b], acc.at[cidx.at[m]], sems[b],
                                     add=True)

                    @pl.when(m > jlo)
                    def _():
                        pltpu.make_async_copy(rows[bp], acc.at[cidx.at[m - 1]],
                                              sems[bp]).wait()

                    nxt = m + NB - 1

                    @pl.when(nxt < jhi)
                    def _():
                        pltpu.async_copy(h_hbm.at[ridx.at[nxt]], rows[bp],
                                         semg[bp])

            return carry

        lax.fori_loop(0, (MAXJ + NB - 1) // NB, group, 0)

        blast = (jhi - 1 - jlo) % NB
        for b in range(NB):
            @pl.when(blast == b)
            def _(b=b):
                pltpu.make_async_copy(rows[b], acc.at[cidx.at[jhi - 1]],
                                      sems[b]).wait()

        plsc.subcore_barrier()

        pltpu.sync_copy(acc.at[pl.ds(s * NPS, NPS)],
                        out_hbm.at[c, pl.ds(s * NPS, NPS)])

    return spmm


@functools.lru_cache(maxsize=None)
def _make_deg():
    D = 16
    mesh = plsc.VectorSubcoreMesh(core_axis_name="c", subcore_axis_name="s")

    @functools.partial(
        pl.kernel,
        out_type=jax.ShapeDtypeStruct((NC, N, D), jnp.float32),
        mesh=mesh,
        compiler_params=pltpu.CompilerParams(use_tc_tiling_on_sc=False),
        scratch_types=[
            pltpu.VMEM((MAXJ, CH), jnp.int32),
            pltpu.VMEM((CH, D), jnp.float32),
            pltpu.VMEM((ZCH, D), jnp.float32),
            pltpu.VMEM_SHARED((N, D), jnp.float32),
            pltpu.SemaphoreType.DMA,
        ],
    )
    def deg(eir_hbm, out_hbm, cidx, ones, zbuf, acc, sems):
        c = lax.axis_index("c")
        s = lax.axis_index("s")
        jlo, jhi, base = _span(s)

        pltpu.sync_copy(eir_hbm.at[1, pl.ds(c * CPC + base, MAXJ)], cidx)

        ov = jnp.ones((16,), jnp.float32)
        zv = jnp.zeros((16,), jnp.float32)

        def orow(i, carry):
            ones[i, pl.ds(0, 16)] = ov
            return carry

        lax.fori_loop(0, CH, orow, 0)

        def zrow(i, carry):
            zbuf[i, pl.ds(0, 16)] = zv
            return carry

        lax.fori_loop(0, ZCH, zrow, 0)
        for i in range(NZPS):
            pltpu.async_copy(zbuf, acc.at[pl.ds(s * NPS + i * ZCH, ZCH)], sems)
        for i in range(NZPS):
            pltpu.make_async_copy(zbuf, acc.at[pl.ds(s * NPS + i * ZCH, ZCH)],
                                  sems).wait()
        plsc.subcore_barrier()

        def fire(m, carry):
            @pl.when(m < jhi)
            def _():
                pltpu.async_copy(ones, acc.at[cidx.at[m]], sems, add=True)

            return carry

        lax.fori_loop(jlo, jlo + MAXJ, fire, 0)

        def drain(m, carry):
            @pl.when(m < jhi)
            def _():
                pltpu.make_async_copy(ones, acc.at[cidx.at[m]], sems).wait()

            return carry

        lax.fori_loop(jlo, jlo + MAXJ, drain, 0)
        plsc.subcore_barrier()

        pltpu.sync_copy(acc.at[pl.ds(s * NPS, NPS)],
                        out_hbm.at[c, pl.ds(s * NPS, NPS)])

    return deg


def _row_spec(d):
    return pl.BlockSpec((BN, d), lambda i: (i, 0))


def _acc_spec(d):
    return pl.BlockSpec((NC, BN, d), lambda i: (0, i, 0))


def _full_spec(r, c):
    return pl.BlockSpec((r, c), lambda i: (0, 0))


def _stage_a0(x, w):
    def body(x_ref, w_ref, o_ref):
        o_ref[...] = jnp.dot(x_ref[...], w_ref[...],
                             preferred_element_type=jnp.float32)

    return pl.pallas_call(
        body,
        grid=(N // BN,),
        in_specs=[_row_spec(256), _full_spec(256, 128)],
        out_specs=_row_spec(128),
        out_shape=jax.ShapeDtypeStruct((N, 128), jnp.float32),
    )(x, w)


def _stage_a1(dacc, xw):
    def body(dacc_ref, xw_ref, dinv_ref, p_ref):
        deg = dacc_ref[0, :, 0:1] + dacc_ref[1, :, 0:1] + 1.0
        dinv = lax.rsqrt(jnp.maximum(deg, 1e-12))
        dinv_ref[...] = dinv
        p_ref[...] = dinv * xw_ref[...]

    return pl.pallas_call(
        body,
        grid=(N // BN,),
        in_specs=[_acc_spec(16), _row_spec(128)],
        out_specs=[_row_spec(1), _row_spec(128)],
        out_shape=[jax.ShapeDtypeStruct((N, 1), jnp.float32),
                   jax.ShapeDtypeStruct((N, 128), jnp.float32)],
    )(dacc, xw)


def _stage_b(acc, p, dinv, b, w):
    def body(acc_ref, p_ref, dinv_ref, b_ref, w_ref, o_ref):
        dinv = dinv_ref[...]
        h = jnp.maximum(dinv * (acc_ref[0] + acc_ref[1] + p_ref[...])
                        + b_ref[...], 0.0)
        o_ref[...] = dinv * jnp.dot(h, w_ref[...],
                                    preferred_element_type=jnp.float32)

    return pl.pallas_call(
        body,
        grid=(N // BN,),
        in_specs=[_acc_spec(128), _row_spec(128), _row_spec(1),
                  _full_spec(1, 128), _full_spec(128, 64)],
        out_specs=_row_spec(64),
        out_shape=jax.ShapeDtypeStruct((N, 64), jnp.float32),
    )(acc, p, dinv, b, w)


def _stage_c(acc, p, dinv, b):
    def body(acc_ref, p_ref, dinv_ref, b_ref, o_ref):
        dinv = dinv_ref[...]
        z = jnp.maximum(dinv * (acc_ref[0] + acc_ref[1] + p_ref[...])
                        + b_ref[...], 0.0)
        o_ref[...] = dinv * z

    return pl.pallas_call(
        body,
        grid=(N // BN,),
        in_specs=[_acc_spec(64), _row_spec(64), _row_spec(1),
                  _full_spec(1, 64)],
        out_specs=_row_spec(64),
        out_shape=jax.ShapeDtypeStruct((N, 64), jnp.float32),
    )(acc, p, dinv, b)


def _stage_d(acc, p, dinv, w_a1, b_a1, w_s, b_s):
    def body(acc_ref, p_ref, dinv_ref, wa_ref, ba_ref, ws_ref, bs_ref,
             struct_ref, p4_ref):
        dinv = dinv_ref[...]
        t = dinv * (acc_ref[0] + acc_ref[1] + p_ref[...])
        struct_ref[...] = jnp.maximum(
            jnp.dot(t, ws_ref[...], preferred_element_type=jnp.float32)
            + bs_ref[...], 0.0)
        a = jnp.maximum(
            jnp.dot(t, wa_ref[...], preferred_element_type=jnp.float32)
            + ba_ref[...], 0.0)
        p4_ref[...] = dinv * a

    return pl.pallas_call(
        body,
        grid=(N // BN,),
        in_specs=[_acc_spec(64), _row_spec(64), _row_spec(1),
                  _full_spec(64, 128), _full_spec(1, 128),
                  _full_spec(64, 64), _full_spec(1, 64)],
        out_specs=[_row_spec(64), _row_spec(128)],
        out_shape=[jax.ShapeDtypeStruct((N, 64), jnp.float32),
                   jax.ShapeDtypeStruct((N, 128), jnp.float32)],
    )(acc, p, dinv, w_a1, b_a1, w_s, b_s)


def _stage_e(acc, p, dinv, w, b):
    def body(acc_ref, p_ref, dinv_ref, w_ref, b_ref, o_ref):
        t = dinv_ref[...] * (acc_ref[0] + acc_ref[1] + p_ref[...])
        o_ref[...] = (jnp.dot(t, w_ref[...],
                              preferred_element_type=jnp.float32)
                      + b_ref[...])

    return pl.pallas_call(
        body,
        grid=(N // BN,),
        in_specs=[_acc_spec(128), _row_spec(128), _row_spec(1),
                  _full_spec(128, 256), _full_spec(1, 256)],
        out_specs=_row_spec(256),
        out_shape=jax.ShapeDtypeStruct((N, 256), jnp.float32),
    )(acc, p, dinv, w, b)


def kernel(x, ei, W_e1, b_e1, W_e2, b_e2, W_a1, b_a1, W_a2, b_a2, W_s, b_s):
    eir = ei.astype(jnp.int32).reshape(2, NC * CPC, CH)

    dacc = _make_deg()(eir)
    xw = _stage_a0(x, W_e1)

    dinv, p1 = _stage_a1(dacc, xw)
    acc1 = _make_spmm(128)(p1, eir)
    p2 = _stage_b(acc1, p1, dinv, b_e1.reshape(1, -1), W_e2)
    acc2 = _make_spmm(64)(p2, eir)
    pz = _stage_c(acc2, p2, dinv, b_e2.reshape(1, -1))
    acc3 = _make_spmm(64)(pz, eir)
    struct, p4 = _stage_d(acc3, pz, dinv, W_a1, b_a1.reshape(1, -1),
                          W_s, b_s.reshape(1, -1))
    acc4 = _make_spmm(128)(p4, eir)
    attr = _stage_e(acc4, p4, dinv, W_a2, b_a2.reshape(1, -1))
    return (attr, struct)


# speedup vs baseline: 28.5726x; 1.1035x over previous
import functools

import jax
import jax.numpy as jnp
from jax import lax
from jax.experimental import pallas as pl
from jax.experimental.pallas import tpu as pltpu
from jax.experimental.pallas import tpu_sc as plsc

N = 10000
E = 160000
NC, NS = 2, 16
CH = 128
EPC = E // NC
CPC = EPC // CH
BN = 2000


MAXJ = CPC // NS + 1
NPS = N // NS
ZCH = 25
NZPS = NPS // ZCH


def _span(s):
    lo = (s * CPC) // NS
    hi = ((s + 1) * CPC) // NS
    base = jnp.minimum(lo, CPC - MAXJ)
    return lo - base, hi - base, base


@functools.lru_cache(maxsize=None)
def _make_spmm(D):
    NB = 2 if D > 64 else 4
    mesh = plsc.VectorSubcoreMesh(core_axis_name="c", subcore_axis_name="s")

    @functools.partial(
        pl.kernel,
        out_type=jax.ShapeDtypeStruct((NC, N, D), jnp.float32),
        mesh=mesh,
        compiler_params=pltpu.CompilerParams(use_tc_tiling_on_sc=False),
        scratch_types=[
            pltpu.VMEM((MAXJ, CH), jnp.int32),
            pltpu.VMEM((MAXJ, CH), jnp.int32),
            [pltpu.VMEM((CH, D), jnp.float32) for _ in range(NB)],
            pltpu.VMEM((ZCH, D), jnp.float32),
            pltpu.VMEM_SHARED((N, D), jnp.float32),
            [pltpu.SemaphoreType.DMA for _ in range(NB)],
            pltpu.SemaphoreType.DMA,
        ],
    )
    def spmm(h_hbm, eir_hbm, out_hbm, ridx, cidx, rows, zbuf, acc,
             semg, semi):
        c = lax.axis_index("c")
        s = lax.axis_index("s")
        jlo, jhi, base = _span(s)

        pltpu.async_copy(eir_hbm.at[0, pl.ds(c * CPC + base, MAXJ)], ridx, semi)
        pltpu.sync_copy(eir_hbm.at[1, pl.ds(c * CPC + base, MAXJ)], cidx)
        pltpu.make_async_copy(eir_hbm.at[0, pl.ds(0, MAXJ)], ridx, semi).wait()

        for b in range(NB):
            m = jlo + b

            @pl.when(m < jhi)
            def _(m=m, b=b):
                pltpu.async_copy(h_hbm.at[ridx.at[m]], rows[b], semg[b])

        zv = jnp.zeros((16,), jnp.float32)

        def zrow(i, carry):
            for d in range(D // 16):
                zbuf[i, pl.ds(d * 16, 16)] = zv
            return carry

        lax.fori_loop(0, ZCH, zrow, 0)
        for i in range(NZPS):
            pltpu.async_copy(zbuf, acc.at[pl.ds(s * NPS + i * ZCH, ZCH)], semi)
        for i in range(NZPS):
            pltpu.make_async_copy(zbuf, acc.at[pl.ds(s * NPS + i * ZCH, ZCH)],
                                  semi).wait()
        plsc.subcore_barrier()

        def group(g, carry):
            for b in range(NB):
                m = jlo + g * NB + b

                @pl.when(m < jhi)
                def _(m=m, b=b):
                    pltpu.make_async_copy(h_hbm.at[ridx.at[m]], rows[b],
                                          semg[b]).wait()
                    pltpu.sync_copy(rows[b], acc.at[cidx.at[m]], add=True)
                    nxt = m + NB

                    @pl.when(nxt < jhi)
                    def _():
                        pltpu.async_copy(h_hbm.at[ridx.at[nxt]], rows[b],
                                         semg[b])

            return carry

        lax.fori_loop(0, (MAXJ + NB - 1) // NB, group, 0)
        plsc.subcore_barrier()

        pltpu.sync_copy(acc.at[pl.ds(s * NPS, NPS)],
                        out_hbm.at[c, pl.ds(s * NPS, NPS)])

    return spmm


@functools.lru_cache(maxsize=None)
def _make_deg():
    D = 16
    mesh = plsc.VectorSubcoreMesh(core_axis_name="c", subcore_axis_name="s")

    @functools.partial(
        pl.kernel,
        out_type=jax.ShapeDtypeStruct((NC, N, D), jnp.float32),
        mesh=mesh,
        compiler_params=pltpu.CompilerParams(use_tc_tiling_on_sc=False),
        scratch_types=[
            pltpu.VMEM((MAXJ, CH), jnp.int32),
            pltpu.VMEM((CH, D), jnp.float32),
            pltpu.VMEM((ZCH, D), jnp.float32),
            pltpu.VMEM_SHARED((N, D), jnp.float32),
            pltpu.SemaphoreType.DMA,
        ],
    )
    def deg(eir_hbm, out_hbm, cidx, ones, zbuf, acc, sems):
        c = lax.axis_index("c")
        s = lax.axis_index("s")
        jlo, jhi, base = _span(s)

        pltpu.sync_copy(eir_hbm.at[1, pl.ds(c * CPC + base, MAXJ)], cidx)

        ov = jnp.ones((16,), jnp.float32)
        zv = jnp.zeros((16,), jnp.float32)

        def orow(i, carry):
            ones[i, pl.ds(0, 16)] = ov
            return carry

        lax.fori_loop(0, CH, orow, 0)

        def zrow(i, carry):
            zbuf[i, pl.ds(0, 16)] = zv
            return carry

        lax.fori_loop(0, ZCH, zrow, 0)
        for i in range(NZPS):
            pltpu.async_copy(zbuf, acc.at[pl.ds(s * NPS + i * ZCH, ZCH)], sems)
        for i in range(NZPS):
            pltpu.make_async_copy(zbuf, acc.at[pl.ds(s * NPS + i * ZCH, ZCH)],
                                  sems).wait()
        plsc.subcore_barrier()

        def fire(m, carry):
            @pl.when(m < jhi)
            def _():
                pltpu.async_copy(ones, acc.at[cidx.at[m]], sems, add=True)

            return carry

        lax.fori_loop(jlo, jlo + MAXJ, fire, 0)

        def drain(m, carry):
            @pl.when(m < jhi)
            def _():
                pltpu.make_async_copy(ones, acc.at[cidx.at[m]], sems).wait()

            return carry

        lax.fori_loop(jlo, jlo + MAXJ, drain, 0)
        plsc.subcore_barrier()

        pltpu.sync_copy(acc.at[pl.ds(s * NPS, NPS)],
                        out_hbm.at[c, pl.ds(s * NPS, NPS)])

    return deg


def _row_spec(d):
    return pl.BlockSpec((BN, d), lambda i: (i, 0))


def _acc_spec(d):
    return pl.BlockSpec((NC, BN, d), lambda i: (0, i, 0))


def _full_spec(r, c):
    return pl.BlockSpec((r, c), lambda i: (0, 0))


def _stage_a0(x, w):
    def body(x_ref, w_ref, o_ref):
        o_ref[...] = jnp.dot(x_ref[...], w_ref[...],
                             preferred_element_type=jnp.float32)

    return pl.pallas_call(
        body,
        grid=(N // BN,),
        in_specs=[_row_spec(256), _full_spec(256, 128)],
        out_specs=_row_spec(128),
        out_shape=jax.ShapeDtypeStruct((N, 128), jnp.float32),
    )(x, w)


def _stage_a1(dacc, xw):
    def body(dacc_ref, xw_ref, dinv_ref, p_ref):
        deg = dacc_ref[0, :, 0:1] + dacc_ref[1, :, 0:1] + 1.0
        dinv = lax.rsqrt(jnp.maximum(deg, 1e-12))
        dinv_ref[...] = dinv
        p_ref[...] = dinv * xw_ref[...]

    return pl.pallas_call(
        body,
        grid=(N // BN,),
        in_specs=[_acc_spec(16), _row_spec(128)],
        out_specs=[_row_spec(1), _row_spec(128)],
        out_shape=[jax.ShapeDtypeStruct((N, 1), jnp.float32),
                   jax.ShapeDtypeStruct((N, 128), jnp.float32)],
    )(dacc, xw)


def _stage_b(acc, p, dinv, b, w):
    def body(acc_ref, p_ref, dinv_ref, b_ref, w_ref, o_ref):
        dinv = dinv_ref[...]
        h = jnp.maximum(dinv * (acc_ref[0] + acc_ref[1] + p_ref[...])
                        + b_ref[...], 0.0)
        o_ref[...] = dinv * jnp.dot(h, w_ref[...],
                                    preferred_element_type=jnp.float32)

    return pl.pallas_call(
        body,
        grid=(N // BN,),
        in_specs=[_acc_spec(128), _row_spec(128), _row_spec(1),
                  _full_spec(1, 128), _full_spec(128, 64)],
        out_specs=_row_spec(64),
        out_shape=jax.ShapeDtypeStruct((N, 64), jnp.float32),
    )(acc, p, dinv, b, w)


def _stage_c(acc, p, dinv, b):
    def body(acc_ref, p_ref, dinv_ref, b_ref, o_ref):
        dinv = dinv_ref[...]
        z = jnp.maximum(dinv * (acc_ref[0] + acc_ref[1] + p_ref[...])
                        + b_ref[...], 0.0)
        o_ref[...] = dinv * z

    return pl.pallas_call(
        body,
        grid=(N // BN,),
        in_specs=[_acc_spec(64), _row_spec(64), _row_spec(1),
                  _full_spec(1, 64)],
        out_specs=_row_spec(64),
        out_shape=jax.ShapeDtypeStruct((N, 64), jnp.float32),
    )(acc, p, dinv, b)


def _stage_d(acc, p, dinv, w_a1, b_a1, w_s, b_s):
    def body(acc_ref, p_ref, dinv_ref, wa_ref, ba_ref, ws_ref, bs_ref,
             struct_ref, p4_ref):
        dinv = dinv_ref[...]
        t = dinv * (acc_ref[0] + acc_ref[1] + p_ref[...])
        struct_ref[...] = jnp.maximum(
            jnp.dot(t, ws_ref[...], preferred_element_type=jnp.float32)
            + bs_ref[...], 0.0)
        a = jnp.maximum(
            jnp.dot(t, wa_ref[...], preferred_element_type=jnp.float32)
            + ba_ref[...], 0.0)
        p4_ref[...] = dinv * a

    return pl.pallas_call(
        body,
        grid=(N // BN,),
        in_specs=[_acc_spec(64), _row_spec(64), _row_spec(1),
                  _full_spec(64, 128), _full_spec(1, 128),
                  _full_spec(64, 64), _full_spec(1, 64)],
        out_specs=[_row_spec(64), _row_spec(128)],
        out_shape=[jax.ShapeDtypeStruct((N, 64), jnp.float32),
                   jax.ShapeDtypeStruct((N, 128), jnp.float32)],
    )(acc, p, dinv, w_a1, b_a1, w_s, b_s)


def _stage_e(acc, p, dinv, w, b):
    def body(acc_ref, p_ref, dinv_ref, w_ref, b_ref, o_ref):
        t = dinv_ref[...] * (acc_ref[0] + acc_ref[1] + p_ref[...])
        o_ref[...] = (jnp.dot(t, w_ref[...],
                              preferred_element_type=jnp.float32)
                      + b_ref[...])

    return pl.pallas_call(
        body,
        grid=(N // BN,),
        in_specs=[_acc_spec(128), _row_spec(128), _row_spec(1),
                  _full_spec(128, 256), _full_spec(1, 256)],
        out_specs=_row_spec(256),
        out_shape=jax.ShapeDtypeStruct((N, 256), jnp.float32),
    )(acc, p, dinv, w, b)


def kernel(x, ei, W_e1, b_e1, W_e2, b_e2, W_a1, b_a1, W_a2, b_a2, W_s, b_s):
    eir = ei.astype(jnp.int32).reshape(2, NC * CPC, CH)

    dacc = _make_deg()(eir)
    xw = _stage_a0(x, W_e1)

    dinv, p1 = _stage_a1(dacc, xw)
    acc1 = _make_spmm(128)(p1, eir)
    p2 = _stage_b(acc1, p1, dinv, b_e1.reshape(1, -1), W_e2)
    acc2 = _make_spmm(64)(p2, eir)
    pz = _stage_c(acc2, p2, dinv, b_e2.reshape(1, -1))
    acc3 = _make_spmm(64)(pz, eir)
    struct, p4 = _stage_d(acc3, pz, dinv, W_a1, b_a1.reshape(1, -1),
                          W_s, b_s.reshape(1, -1))
    acc4 = _make_spmm(128)(p4, eir)
    attr = _stage_e(acc4, p4, dinv, W_a2, b_a2.reshape(1, -1))
    return (attr, struct)
